# Initial kernel scaffold; baseline (speedup 1.0000x reference)
import jax
import jax.numpy as jnp
from jax.experimental import pallas as pl


def kernel(grid_features, mesh_features, mesh2mesh_edge_features, grid2mesh_edge_features, grid2mesh_edge_indices, mesh2grid_edge_features, mesh2grid_edge_indices, mesh2mesh_edge_indices, params):
    raise NotImplementedError("write your pallas kernel here")


# speedup vs baseline: 1.3309x; 1.3309x over previous
import functools

import jax
import jax.numpy as jnp
from jax import lax
from jax.experimental import pallas as pl
from jax.experimental.pallas import tpu as pltpu
from jax.experimental.pallas import tpu_sc as plsc

D = 512
BN = 512
C = 128
CS = 64
NODE_WIN = 2040
BUF_ROWS = 2048
NC = 2
NS = 16
NW = NC * NS


def _pad_rows(x, m):
  n = x.shape[0]
  p = (-n) % m
  if p == 0:
    return x
  return jnp.concatenate([x, jnp.zeros((p,) + x.shape[1:], x.dtype)], axis=0)


def _pad_cols(x, m):
  c = x.shape[1]
  p = (-c) % m
  if p == 0:
    return x
  return jnp.concatenate([x, jnp.zeros((x.shape[0], p), x.dtype)], axis=1)


def _mlp_body(nx, na, has_ln, has_res, *refs):
  xs = refs[:nx]
  ws = refs[nx:2 * nx]
  k = 2 * nx
  ads = refs[k:k + na]
  k += na
  b1r, w2r, b2r = refs[k], refs[k + 1], refs[k + 2]
  k += 3
  if has_ln:
    lsr, lbr = refs[k], refs[k + 1]
    k += 2
  if has_res:
    rr = refs[k]
    k += 1
  outr = refs[k]
  acc = jnp.dot(xs[0][...].astype(ws[0].dtype), ws[0][...],
                preferred_element_type=jnp.float32)
  for j in range(1, nx):
    acc += jnp.dot(xs[j][...].astype(ws[j].dtype), ws[j][...],
                   preferred_element_type=jnp.float32)
  for a in ads:
    acc += a[...].astype(jnp.float32)
  acc += b1r[...]
  h = acc * jax.nn.sigmoid(acc)
  y = jnp.dot(h.astype(w2r.dtype), w2r[...],
              preferred_element_type=jnp.float32) + b2r[...]
  if has_ln:
    mu = jnp.mean(y, axis=-1, keepdims=True)
    var = jnp.mean((y - mu) * (y - mu), axis=-1, keepdims=True)
    y = (y - mu) * lax.rsqrt(var + 1e-5) * lsr[...] + lbr[...]
  if has_res:
    y = y + rr[...]
  outr[...] = y


def _mlp_tc(xs, w1s, b1, w2, b2, adds=(), ln=None, res=None):
  n = (xs[0] if xs else adds[0]).shape[0]
  assert n % BN == 0, n
  dh = b1.shape[0]
  dout = w2.shape[1]
  nx = len(xs)
  na = len(adds)
  has_ln = ln is not None
  has_res = res is not None

  in_specs = []
  args = []
  for x in xs:
    in_specs.append(pl.BlockSpec((BN, x.shape[1]), lambda i: (i, 0)))
    args.append(x)
  for w in w1s:
    in_specs.append(pl.BlockSpec(w.shape, lambda i: (0, 0)))
    args.append(w)
  for a in adds:
    in_specs.append(pl.BlockSpec((BN, dh), lambda i: (i, 0)))
    args.append(a)
  for shp in ((1, dh), (dh, dout), (1, dout)):
    in_specs.append(pl.BlockSpec(shp, lambda i: (0, 0)))
  args += [b1.reshape(1, dh), w2, b2.reshape(1, dout)]
  if has_ln:
    ls, lb = ln
    in_specs.append(pl.BlockSpec((1, dout), lambda i: (0, 0)))
    in_specs.append(pl.BlockSpec((1, dout), lambda i: (0, 0)))
    args += [ls.reshape(1, dout), lb.reshape(1, dout)]
  if has_res:
    in_specs.append(pl.BlockSpec((BN, dout), lambda i: (i, 0)))
    args.append(res)

  return pl.pallas_call(
      functools.partial(_mlp_body, nx, na, has_ln, has_res),
      grid=(n // BN,),
      in_specs=in_specs,
      out_specs=pl.BlockSpec((BN, dout), lambda i: (i, 0)),
      out_shape=jax.ShapeDtypeStruct((n, dout), jnp.float32),
      compiler_params=pltpu.CompilerParams(
          dimension_semantics=("arbitrary",)),
  )(*args)


def _linear_body(x_ref, w_ref, o_ref):
  o_ref[...] = jnp.dot(x_ref[...].astype(w_ref.dtype), w_ref[...],
                       preferred_element_type=jnp.float32
                       ).astype(o_ref.dtype)


def _linear_tc(x, w):
  n, kdim = x.shape
  dout = w.shape[1]
  assert n % BN == 0
  return pl.pallas_call(
      _linear_body,
      grid=(n // BN,),
      in_specs=[pl.BlockSpec((BN, kdim), lambda i: (i, 0)),
                pl.BlockSpec((kdim, dout), lambda i: (0, 0))],
      out_specs=pl.BlockSpec((BN, dout), lambda i: (i, 0)),
      out_shape=jax.ShapeDtypeStruct((n, dout), jnp.float32),
      compiler_params=pltpu.CompilerParams(
          dimension_semantics=("arbitrary",)),
  )(x, w)


@functools.lru_cache(maxsize=None)
def _sc_gather_fn(t_rows, d_cols, b_rows, dtype_name):
  dtype = jnp.dtype(dtype_name)
  nch = b_rows // C
  assert b_rows % C == 0
  mesh = plsc.VectorSubcoreMesh(core_axis_name="c", subcore_axis_name="s")

  @functools.partial(
      pl.kernel,
      mesh=mesh,
      out_type=jax.ShapeDtypeStruct((b_rows, d_cols), dtype),
      scratch_types=[
          pltpu.VMEM((C,), jnp.int32),
          pltpu.VMEM((C, d_cols), dtype),
          pltpu.SemaphoreType.DMA,
      ],
      compiler_params=pltpu.CompilerParams(needs_layout_passes=False),
  )
  def gk(table, idxh, out, idxv, rows, sem):
    cc = lax.axis_index("c")
    ss = lax.axis_index("s")
    wid = ss * NC + cc
    n_my = nch // NW + jnp.where(wid < nch % NW, 1, 0)

    def body(t, carry):
      j = wid + t * NW
      base = j * C
      pltpu.sync_copy(idxh.at[pl.ds(base, C)], idxv)
      pltpu.async_copy(table.at[idxv], rows, sem).wait()
      pltpu.sync_copy(rows, out.at[pl.ds(base, C)])
      return carry

    lax.fori_loop(0, n_my, body, 0)

  return gk


def _sc_gather(table, idx):
  return _sc_gather_fn(table.shape[0], table.shape[1], idx.shape[0],
                       table.dtype.name)(table, idx)


WIN_N = 160


def _scatter_plan(d_sorted, n_nodes):
  e_pad = d_sorted.shape[0]
  e_plan = e_pad - 64
  nwin = -(-(n_nodes + 1) // WIN_N)
  nwin_pad = -(-nwin // 16) * 16

  d_plan = d_sorted[:e_plan]
  bounds = jnp.arange(nwin + 1, dtype=jnp.int32) * WIN_N
  eoff = jnp.searchsorted(d_plan, bounds).astype(jnp.int32)
  lo, hi = eoff[:-1], eoff[1:]
  off8 = (lo // 8) * 8
  nck = jnp.where(hi > lo, -(-(hi - off8) // CS), 0).astype(jnp.int32)

  winid = (d_plan // WIN_N).astype(jnp.int32)
  soff = (d_plan - winid * WIN_N).astype(jnp.int32)
  winid = jnp.concatenate([winid, jnp.full((64,), nwin + 1, jnp.int32)])
  soff = jnp.concatenate([soff, jnp.zeros((64,), jnp.int32)])

  def padp(a):
    return jnp.concatenate([a, jnp.zeros((nwin_pad - nwin,), jnp.int32)])

  meta = jnp.concatenate([padp(off8), padp(nck)])
  return soff, winid, meta, nwin


@functools.lru_cache(maxsize=None)
def _sc_scatter_fn(e_pad, nwin):
  nwin_pad = -(-nwin // 16) * 16
  rounds = -(-nwin // NW)
  mesh = plsc.VectorSubcoreMesh(core_axis_name="c", subcore_axis_name="s")

  @functools.partial(
      pl.kernel,
      mesh=mesh,
      out_type=jax.ShapeDtypeStruct((nwin * WIN_N, D), jnp.float32),
      scratch_types=[
          pltpu.VMEM((CS, D), jnp.float32),
          pltpu.VMEM((CS,), jnp.int32),
          pltpu.VMEM((CS,), jnp.int32),
          pltpu.VMEM((WIN_N + 8, D), jnp.float32),
          pltpu.VMEM((2 * nwin_pad,), jnp.int32),
      ],
      compiler_params=pltpu.CompilerParams(needs_layout_passes=False),
  )
  def sk(e_hbm, soff_hbm, winid_hbm, meta_hbm, out_hbm,
         ebuf, sv, wv, acc, metav):
    cc = lax.axis_index("c")
    ss = lax.axis_index("s")
    wid = ss * NC + cc
    pltpu.sync_copy(meta_hbm, metav)
    lanes = lax.iota(jnp.int32, 16)

    def meta_at(k):
      r = jnp.int32(0)
      for w in range(2 * nwin_pad // 16):
        vec = metav[pl.ds(w * 16, 16)]
        sel = jnp.where(lanes + (w * 16) == k, vec, jnp.int32(0))
        r = jnp.maximum(r, jnp.max(sel, axis=0))
      return r

    def round_body(rr, carry):
      w = rr * NW + wid
      live = w < nwin

      @pl.when(live)
      def _window():
        off8 = meta_at(w)
        nck = meta_at(nwin_pad + w)

        def zrow(i, c1):
          def zcol(j, c2):
            acc[i, pl.ds(j * 16, 16)] = jnp.zeros((16,), jnp.float32)
            return c2
          return lax.fori_loop(0, D // 16, zcol, c1)

        lax.fori_loop(0, WIN_N + 8, zrow, 0)

        def chunk(k, c1):
          base = pl.multiple_of(off8 + CS * k, 8)
          pltpu.sync_copy(e_hbm.at[pl.ds(base, CS)], ebuf)
          pltpu.sync_copy(soff_hbm.at[pl.ds(base, CS)], sv)
          pltpu.sync_copy(winid_hbm.at[pl.ds(base, CS)], wv)
          for g in range(CS // 16):
            offv = sv[pl.ds(g * 16, 16)]
            wvv = wv[pl.ds(g * 16, 16)]

            def lane_body(ln, c2):
              slot = jnp.max(jnp.where(lanes == ln, offv, 0), axis=0)
              we = jnp.max(jnp.where(lanes == ln, wvv, 0), axis=0)
              slot = jnp.where(we == w, slot, jnp.int32(WIN_N))
              erow = g * 16 + ln
              for cj in range(D // 16):
                plsc.addupdate(acc.at[slot, pl.ds(cj * 16, 16)],
                               ebuf[erow, pl.ds(cj * 16, 16)])
              return c2

            lax.fori_loop(0, 16, lane_body, 0)
          return c1

        lax.fori_loop(0, nck, chunk, 0)
        pltpu.sync_copy(acc.at[pl.ds(0, WIN_N)],
                        out_hbm.at[pl.ds(pl.multiple_of(w * WIN_N, 8), WIN_N)])

      return carry

    lax.fori_loop(0, rounds, round_body, 0)

  return sk


def _sc_segment_sum(e_sorted, soff, winid, meta, nwin):
  return _sc_scatter_fn(e_sorted.shape[0], nwin)(
      e_sorted, soff, winid, meta)


def _prep_edges(eidx, n_src, n_dst, e_pad):
  s = eidx[0].astype(jnp.int32)
  d = eidx[1].astype(jnp.int32)
  e = d.shape[0]
  perm = jnp.argsort(d).astype(jnp.int32)
  pad = e_pad - e
  perm_pad = jnp.concatenate([perm, jnp.zeros((pad,), jnp.int32)])
  s_pad = jnp.concatenate([s[perm], jnp.zeros((pad,), jnp.int32)])
  d_pad = jnp.concatenate(
      [d[perm], jnp.full((pad,), n_dst, jnp.int32)])
  return perm_pad, s_pad, d_pad


def _bf16(x):
  return x.astype(jnp.bfloat16)


def kernel(grid_features, mesh_features, mesh2mesh_edge_features,
           grid2mesh_edge_features, grid2mesh_edge_indices,
           mesh2grid_edge_features, mesh2grid_edge_indices,
           mesh2mesh_edge_indices, params):
  p = params
  ng, f = grid_features.shape
  nm = mesh_features.shape[0]
  em = mesh2mesh_edge_features.shape[0]
  eg = grid2mesh_edge_features.shape[0]
  emg = mesh2grid_edge_features.shape[0]

  ng_pad = -(-ng // BN) * BN
  nm_pad = -(-nm // BN) * BN
  em_pad = -(-(em + 64) // BN) * BN
  eg_pad = -(-(eg + 64) // BN) * BN
  emg_pad = -(-(emg + 64) // BN) * BN

  g2m_perm, g2m_s, g2m_d = _prep_edges(grid2mesh_edge_indices, ng, nm, eg_pad)
  m2g_perm, m2g_s, m2g_d = _prep_edges(mesh2grid_edge_indices, nm, ng, emg_pad)
  m2m_perm, m2m_s, m2m_d = _prep_edges(mesh2mesh_edge_indices, nm, nm, em_pad)
  g2m_plan = _scatter_plan(g2m_d, nm)
  m2g_plan = _scatter_plan(m2g_d, ng)
  m2m_plan = _scatter_plan(m2m_d, nm)

  def enc(q, x):
    return _mlp_tc([x], [_bf16(_pad_rows(q["W1"], C))], q["b1"],
                   _bf16(q["W2"]), q["b2"], ln=(q["ls"], q["lb"]))

  gf = _pad_rows(_pad_cols(grid_features, C), BN)
  mf = _pad_rows(_pad_cols(mesh_features, C), BN)
  g = enc(p["enc_grid"], gf)
  m = enc(p["enc_mesh"], mf)

  def enc_edges(q, feats, perm):
    f_sorted = _sc_gather(_pad_cols(feats, C), perm)
    return enc(q, f_sorted)

  e_m2m = enc_edges(p["enc_m2m"], mesh2mesh_edge_features, m2m_perm)
  e_g2m = enc_edges(p["enc_g2m"], grid2mesh_edge_features, g2m_perm)
  e_m2g = enc_edges(p["enc_m2g"], mesh2grid_edge_features, m2g_perm)

  def interaction(e, src_tab, dst_tab, s_idx, d_idx, q):
    w1 = q["W1"]
    ps = _linear_tc(src_tab, _bf16(w1[D:2 * D]))
    pd = _linear_tc(dst_tab, _bf16(w1[2 * D:]))
    g1 = _sc_gather(ps, s_idx)
    g2 = _sc_gather(pd, d_idx)
    return _mlp_tc([e], [_bf16(w1[:D])],
                   q["b1"], _bf16(q["W2"]), q["b2"], adds=[g1, g2],
                   ln=(q["ls"], q["lb"]), res=e)

  def node_update(x, agg, q):
    w1 = q["W1"]
    return _mlp_tc([x, agg], [_bf16(w1[:D]), _bf16(w1[D:])],
                   q["b1"], _bf16(q["W2"]), q["b2"],
                   ln=(q["ls"], q["lb"]), res=x)

  e = interaction(e_g2m, g, m, g2m_s, g2m_d, p["g2m_edge"])
  agg = _pad_rows(_sc_segment_sum(e, *g2m_plan), BN)[:nm_pad]
  m = node_update(m, agg, p["g2m_mesh"])
  q = p["g2m_grid"]
  g = _mlp_tc([g], [_bf16(q["W1"])], q["b1"], _bf16(q["W2"]), q["b2"],
              ln=(q["ls"], q["lb"]), res=g)

  for lp in p["proc"]:
    e_m2m = interaction(e_m2m, m, m, m2m_s, m2m_d, lp["edge"])
    agg = _pad_rows(_sc_segment_sum(e_m2m, *m2m_plan), BN)[:nm_pad]
    m = node_update(m, agg, lp["node"])

  e = interaction(e_m2g, m, g, m2g_s, m2g_d, p["m2g_edge"])
  agg = _pad_rows(_sc_segment_sum(e, *m2g_plan), BN)[:ng_pad]
  g = node_update(g, agg, p["m2g_grid"])

  q = p["dec"]
  out = _mlp_tc([g], [_bf16(q["W1"])], q["b1"],
                _bf16(_pad_cols(q["W2"], C)),
                jnp.concatenate([q["b2"], jnp.zeros((C - f,), jnp.float32)]))
  return out[:ng, :f]
